# baseline (speedup 1.0000x reference)
import numpy as np
import jax
import jax.numpy as jnp
from jax.experimental import pallas as pl
from jax.experimental.pallas import tpu as pltpu

T = 4096
NCH = 256
FREQ = T // 2 + 1
FP = 2304
TOPK = 20
CHUNK = 128
NCHUNK = FP // CHUNK


def _basis_np():
    f = np.arange(FP, dtype=np.float64)
    t = np.arange(T, dtype=np.float64)
    ft = np.outer(f, t)
    ang = np.mod(ft, T) * (2.0 * np.pi / T)
    c = np.cos(ang)
    s = np.sin(ang)
    c[FREQ:, :] = 0.0
    s[FREQ:, :] = 0.0
    w = np.full((FP, 1), 2.0 / T)
    w[0, 0] = 1.0 / T
    w[T // 2, 0] = 1.0 / T
    w[FREQ:, 0] = 0.0
    ct = c.astype(np.float32)
    st = s.astype(np.float32)
    cth = ct.astype(jnp.bfloat16)
    ctl = (ct - np.asarray(cth, np.float32)).astype(jnp.bfloat16)
    sth = st.astype(jnp.bfloat16)
    stl = (st - np.asarray(sth, np.float32)).astype(jnp.bfloat16)
    ic = np.ascontiguousarray((c * w).T).astype(jnp.bfloat16)
    isn = np.ascontiguousarray((s * w).T).astype(jnp.bfloat16)
    return cth, ctl, sth, stl, ic, isn


_CTH, _CTL, _STH, _STL, _IC, _ISN = _basis_np()


def _fwd_kernel(cth_ref, ctl_ref, sth_ref, stl_ref, x_ref, xre_ref, xim_ref):
    xb = x_ref[0]
    xh = xb.astype(jnp.bfloat16)
    xl = (xb - xh.astype(jnp.float32)).astype(jnp.bfloat16)

    def mm3(h_ref, l_ref):
        acc = jnp.dot(h_ref[...], xh, preferred_element_type=jnp.float32)
        acc += jnp.dot(h_ref[...], xl, preferred_element_type=jnp.float32)
        acc += jnp.dot(l_ref[...], xh, preferred_element_type=jnp.float32)
        return acc

    xre_ref[0] = mm3(cth_ref, ctl_ref)
    xim_ref[0] = mm3(sth_ref, stl_ref)


NPICK = 4
NSWEEP = TOPK // NPICK


def _chunk_top4(v):
    out = []
    for _ in range(NPICK):
        m = jnp.max(v, axis=0, keepdims=True)
        out.append(m)
        v = jnp.where(v >= m, -1.0, v)
    return out


def _merge4(a, b):
    e = [jnp.maximum(a[i], b[NPICK - 1 - i]) for i in range(NPICK)]
    lo02, hi02 = jnp.minimum(e[0], e[2]), jnp.maximum(e[0], e[2])
    lo13, hi13 = jnp.minimum(e[1], e[3]), jnp.maximum(e[1], e[3])
    return [jnp.maximum(hi02, hi13), jnp.minimum(hi02, hi13),
            jnp.maximum(lo02, lo13), jnp.minimum(lo02, lo13)]


def _topk_kernel(xre_ref, xim_ref, yre_ref, yim_ref, mag_scr):
    m = None
    for c in range(NCHUNK):
        sl = pl.ds(c * CHUNK, CHUNK)
        xr = xre_ref[0, sl, :]
        xi = xim_ref[0, sl, :]
        v = xr * xr + xi * xi
        mag_scr[sl, :] = v
        t = _chunk_top4(v)
        m = t if m is None else _merge4(m, t)

    for s in range(NSWEEP):
        thr = m[NPICK - 1]
        nm = None
        for c in range(NCHUNK):
            sl = pl.ds(c * CHUNK, CHUNK)
            mg = mag_scr[sl, :]
            pick = mg >= thr
            mgu = jnp.where(pick, -1.0, mg)
            mag_scr[sl, :] = mgu
            if s + 1 < NSWEEP:
                t = _chunk_top4(mgu)
                nm = t if nm is None else _merge4(nm, t)
        m = nm

    for c in range(NCHUNK):
        sl = pl.ds(c * CHUNK, CHUNK)
        ws = jnp.where(mag_scr[sl, :] < 0.0, 1.0, 0.0)
        yre_ref[0, sl, :] = (xre_ref[0, sl, :] * ws).astype(jnp.bfloat16)
        yim_ref[0, sl, :] = (xim_ref[0, sl, :] * ws).astype(jnp.bfloat16)


def _inv_kernel(ic_ref, isn_ref, yre_ref, yim_ref, x_ref, o_ref):
    filt = jnp.dot(ic_ref[...], yre_ref[0], preferred_element_type=jnp.float32)
    filt = filt + jnp.dot(isn_ref[...], yim_ref[0], preferred_element_type=jnp.float32)
    o_ref[0] = x_ref[0] - filt


def kernel(batch_x, W1, b1, W2, b2, W3, b3):
    B = batch_x.shape[0]
    cth = jnp.asarray(_CTH)
    ctl = jnp.asarray(_CTL)
    sth = jnp.asarray(_STH)
    stl = jnp.asarray(_STL)
    ic = jnp.asarray(_IC)
    isn = jnp.asarray(_ISN)

    FH = FP // 4
    xre, xim = pl.pallas_call(
        _fwd_kernel,
        grid=(4, B),
        in_specs=[
            pl.BlockSpec((FH, T), lambda h, b: (h, 0)),
            pl.BlockSpec((FH, T), lambda h, b: (h, 0)),
            pl.BlockSpec((FH, T), lambda h, b: (h, 0)),
            pl.BlockSpec((FH, T), lambda h, b: (h, 0)),
            pl.BlockSpec((1, T, NCH), lambda h, b: (b, 0, 0)),
        ],
        out_specs=[
            pl.BlockSpec((1, FH, NCH), lambda h, b: (b, h, 0)),
            pl.BlockSpec((1, FH, NCH), lambda h, b: (b, h, 0)),
        ],
        out_shape=[jax.ShapeDtypeStruct((B, FP, NCH), jnp.float32)] * 2,
    )(cth, ctl, sth, stl, batch_x)

    yre, yim = pl.pallas_call(
        _topk_kernel,
        grid=(B,),
        in_specs=[
            pl.BlockSpec((1, FP, NCH), lambda b: (b, 0, 0)),
            pl.BlockSpec((1, FP, NCH), lambda b: (b, 0, 0)),
        ],
        out_specs=[
            pl.BlockSpec((1, FP, NCH), lambda b: (b, 0, 0)),
            pl.BlockSpec((1, FP, NCH), lambda b: (b, 0, 0)),
        ],
        out_shape=[jax.ShapeDtypeStruct((B, FP, NCH), jnp.bfloat16)] * 2,
        scratch_shapes=[pltpu.VMEM((FP, NCH), jnp.float32)],
    )(xre, xim)

    TH = T // 4
    out = pl.pallas_call(
        _inv_kernel,
        grid=(4, B),
        in_specs=[
            pl.BlockSpec((TH, FP), lambda h, b: (h, 0)),
            pl.BlockSpec((TH, FP), lambda h, b: (h, 0)),
            pl.BlockSpec((1, FP, NCH), lambda h, b: (b, 0, 0)),
            pl.BlockSpec((1, FP, NCH), lambda h, b: (b, 0, 0)),
            pl.BlockSpec((1, TH, NCH), lambda h, b: (b, h, 0)),
        ],
        out_specs=pl.BlockSpec((1, TH, NCH), lambda h, b: (b, h, 0)),
        out_shape=jax.ShapeDtypeStruct((B, T, NCH), jnp.float32),
    )(ic, isn, yre, yim, batch_x)
    return out


# speedup vs baseline: 1.0723x; 1.0723x over previous
import numpy as np
import jax
import jax.numpy as jnp
from jax.experimental import pallas as pl
from jax.experimental.pallas import tpu as pltpu

T = 4096
NCH = 256
FREQ = T // 2 + 1
FP = 2176
TOPK = 20
CHUNK = 128
NCHUNK = FP // CHUNK


def _basis_np():
    f = np.arange(FP, dtype=np.float64)
    t = np.arange(T, dtype=np.float64)
    ft = np.outer(f, t)
    ang = np.mod(ft, T) * (2.0 * np.pi / T)
    c = np.cos(ang)
    s = np.sin(ang)
    c[FREQ:, :] = 0.0
    s[FREQ:, :] = 0.0
    w = np.full((FP, 1), 2.0 / T)
    w[0, 0] = 1.0 / T
    w[T // 2, 0] = 1.0 / T
    w[FREQ:, 0] = 0.0
    ct = c.astype(np.float32)
    st = s.astype(np.float32)
    cth = ct.astype(jnp.bfloat16)
    ctl = (ct - np.asarray(cth, np.float32)).astype(jnp.bfloat16)
    sth = st.astype(jnp.bfloat16)
    stl = (st - np.asarray(sth, np.float32)).astype(jnp.bfloat16)
    ic = np.ascontiguousarray((c * w).T).astype(jnp.bfloat16)
    isn = np.ascontiguousarray((s * w).T).astype(jnp.bfloat16)
    return cth, ctl, sth, stl, ic, isn


_CTH, _CTL, _STH, _STL, _IC, _ISN = _basis_np()


def _fwd_kernel(cth_ref, ctl_ref, sth_ref, stl_ref, x_ref, xre_ref, xim_ref):
    xb = x_ref[0]
    xh = xb.astype(jnp.bfloat16)
    xl = (xb - xh.astype(jnp.float32)).astype(jnp.bfloat16)

    def mm3(h_ref, l_ref):
        acc = jnp.dot(h_ref[...], xh, preferred_element_type=jnp.float32)
        acc += jnp.dot(h_ref[...], xl, preferred_element_type=jnp.float32)
        acc += jnp.dot(l_ref[...], xh, preferred_element_type=jnp.float32)
        return acc

    xre_ref[0] = mm3(cth_ref, ctl_ref)
    xim_ref[0] = mm3(sth_ref, stl_ref)


def _topk_kernel(xre_ref, xim_ref, yre_ref, yim_ref, mag_scr):
    for c in range(NCHUNK):
        sl = pl.ds(c * CHUNK, CHUNK)
        xr = xre_ref[0, sl, :]
        xi = xim_ref[0, sl, :]
        mag_scr[sl, :] = xr * xr + xi * xi

    m = jnp.max(mag_scr[pl.ds(0, CHUNK), :], axis=0, keepdims=True)
    for c in range(1, NCHUNK):
        m = jnp.maximum(
            m, jnp.max(mag_scr[pl.ds(c * CHUNK, CHUNK), :], axis=0, keepdims=True))

    def body(_, m):
        nm = jnp.full((1, NCH), -2.0, dtype=jnp.float32)
        for c in range(NCHUNK):
            sl = pl.ds(c * CHUNK, CHUNK)
            mg = mag_scr[sl, :]
            pick = mg >= m
            mgu = jnp.where(pick, -1.0, mg)
            mag_scr[sl, :] = mgu
            nm = jnp.maximum(nm, jnp.max(mgu, axis=0, keepdims=True))
        return nm

    jax.lax.fori_loop(0, TOPK, body, m)

    for c in range(NCHUNK):
        sl = pl.ds(c * CHUNK, CHUNK)
        ws = jnp.where(mag_scr[sl, :] < 0.0, 1.0, 0.0)
        yre_ref[0, sl, :] = (xre_ref[0, sl, :] * ws).astype(jnp.bfloat16)
        yim_ref[0, sl, :] = (xim_ref[0, sl, :] * ws).astype(jnp.bfloat16)


def _inv_kernel(ic_ref, isn_ref, yre_ref, yim_ref, x_ref, o_ref):
    filt = jnp.dot(ic_ref[...], yre_ref[0], preferred_element_type=jnp.float32)
    filt = filt + jnp.dot(isn_ref[...], yim_ref[0], preferred_element_type=jnp.float32)
    o_ref[0] = x_ref[0] - filt


def kernel(batch_x, W1, b1, W2, b2, W3, b3):
    B = batch_x.shape[0]
    cth = jnp.asarray(_CTH)
    ctl = jnp.asarray(_CTL)
    sth = jnp.asarray(_STH)
    stl = jnp.asarray(_STL)
    ic = jnp.asarray(_IC)
    isn = jnp.asarray(_ISN)

    FH = FP // 4
    xre, xim = pl.pallas_call(
        _fwd_kernel,
        grid=(4, B),
        in_specs=[
            pl.BlockSpec((FH, T), lambda h, b: (h, 0)),
            pl.BlockSpec((FH, T), lambda h, b: (h, 0)),
            pl.BlockSpec((FH, T), lambda h, b: (h, 0)),
            pl.BlockSpec((FH, T), lambda h, b: (h, 0)),
            pl.BlockSpec((1, T, NCH), lambda h, b: (b, 0, 0)),
        ],
        out_specs=[
            pl.BlockSpec((1, FH, NCH), lambda h, b: (b, h, 0)),
            pl.BlockSpec((1, FH, NCH), lambda h, b: (b, h, 0)),
        ],
        out_shape=[jax.ShapeDtypeStruct((B, FP, NCH), jnp.float32)] * 2,
    )(cth, ctl, sth, stl, batch_x)

    yre, yim = pl.pallas_call(
        _topk_kernel,
        grid=(B,),
        in_specs=[
            pl.BlockSpec((1, FP, NCH), lambda b: (b, 0, 0)),
            pl.BlockSpec((1, FP, NCH), lambda b: (b, 0, 0)),
        ],
        out_specs=[
            pl.BlockSpec((1, FP, NCH), lambda b: (b, 0, 0)),
            pl.BlockSpec((1, FP, NCH), lambda b: (b, 0, 0)),
        ],
        out_shape=[jax.ShapeDtypeStruct((B, FP, NCH), jnp.bfloat16)] * 2,
        scratch_shapes=[pltpu.VMEM((FP, NCH), jnp.float32)],
    )(xre, xim)

    TH = T // 2
    out = pl.pallas_call(
        _inv_kernel,
        grid=(2, B),
        in_specs=[
            pl.BlockSpec((TH, FP), lambda h, b: (h, 0)),
            pl.BlockSpec((TH, FP), lambda h, b: (h, 0)),
            pl.BlockSpec((1, FP, NCH), lambda h, b: (b, 0, 0)),
            pl.BlockSpec((1, FP, NCH), lambda h, b: (b, 0, 0)),
            pl.BlockSpec((1, TH, NCH), lambda h, b: (b, h, 0)),
        ],
        out_specs=pl.BlockSpec((1, TH, NCH), lambda h, b: (b, h, 0)),
        out_shape=jax.ShapeDtypeStruct((B, T, NCH), jnp.float32),
    )(ic, isn, yre, yim, batch_x)
    return out


# speedup vs baseline: 1.0749x; 1.0024x over previous
import numpy as np
import jax
import jax.numpy as jnp
from jax.experimental import pallas as pl
from jax.experimental.pallas import tpu as pltpu

T = 4096
NCH = 256
FREQ = T // 2 + 1
FP = 2176
TOPK = 20
CHUNK = 128
NCHUNK = FP // CHUNK


def _basis_np():
    f = np.arange(FP, dtype=np.float64)
    t = np.arange(T, dtype=np.float64)
    ft = np.outer(f, t)
    ang = np.mod(ft, T) * (2.0 * np.pi / T)
    c = np.cos(ang)
    s = np.sin(ang)
    c[FREQ:, :] = 0.0
    s[FREQ:, :] = 0.0
    w = np.full((FP, 1), 2.0 / T)
    w[0, 0] = 1.0 / T
    w[T // 2, 0] = 1.0 / T
    w[FREQ:, 0] = 0.0
    ct = c.astype(np.float32)
    st = s.astype(np.float32)
    cth = ct.astype(jnp.bfloat16)
    ctl = (ct - np.asarray(cth, np.float32)).astype(jnp.bfloat16)
    sth = st.astype(jnp.bfloat16)
    stl = (st - np.asarray(sth, np.float32)).astype(jnp.bfloat16)
    ic = np.ascontiguousarray((c * w).T).astype(jnp.bfloat16)
    isn = np.ascontiguousarray((s * w).T).astype(jnp.bfloat16)
    return cth, ctl, sth, stl, ic, isn


_CTH, _CTL, _STH, _STL, _IC, _ISN = _basis_np()


def _fwd_kernel(cth_ref, ctl_ref, sth_ref, stl_ref, x_ref, xre_ref, xim_ref):
    xb = x_ref[0]
    xh = xb.astype(jnp.bfloat16)
    xl = (xb - xh.astype(jnp.float32)).astype(jnp.bfloat16)

    def mm3(h_ref, l_ref):
        acc = jnp.dot(h_ref[...], xh, preferred_element_type=jnp.float32)
        acc += jnp.dot(h_ref[...], xl, preferred_element_type=jnp.float32)
        acc += jnp.dot(l_ref[...], xh, preferred_element_type=jnp.float32)
        return acc

    xre_ref[0] = mm3(cth_ref, ctl_ref)
    xim_ref[0] = mm3(sth_ref, stl_ref)


def _topk_kernel(xre_ref, xim_ref, yre_ref, yim_ref, mag_scr):
    for c in range(NCHUNK):
        sl = pl.ds(c * CHUNK, CHUNK)
        xr = xre_ref[0, sl, :]
        xi = xim_ref[0, sl, :]
        mag_scr[sl, :] = xr * xr + xi * xi

    def top2(v):
        t1 = jnp.max(v, axis=0, keepdims=True)
        t2 = jnp.max(jnp.where(v >= t1, -1.0, v), axis=0, keepdims=True)
        return t1, t2

    def merge2(a, b):
        e1 = jnp.maximum(a[0], b[1])
        e2 = jnp.maximum(a[1], b[0])
        return jnp.maximum(e1, e2), jnp.minimum(e1, e2)

    m = None
    for c in range(NCHUNK):
        t = top2(mag_scr[pl.ds(c * CHUNK, CHUNK), :])
        m = t if m is None else merge2(m, t)

    for s in range(TOPK // 2):
        thr = m[1]
        nm = None
        for c in range(NCHUNK):
            sl = pl.ds(c * CHUNK, CHUNK)
            mg = mag_scr[sl, :]
            pick = mg >= thr
            mgu = jnp.where(pick, -1.0, mg)
            mag_scr[sl, :] = mgu
            if s + 1 < TOPK // 2:
                t = top2(mgu)
                nm = t if nm is None else merge2(nm, t)
        m = nm

    for c in range(NCHUNK):
        sl = pl.ds(c * CHUNK, CHUNK)
        ws = jnp.where(mag_scr[sl, :] < 0.0, 1.0, 0.0)
        yre_ref[0, sl, :] = (xre_ref[0, sl, :] * ws).astype(jnp.bfloat16)
        yim_ref[0, sl, :] = (xim_ref[0, sl, :] * ws).astype(jnp.bfloat16)


def _inv_kernel(ic_ref, isn_ref, yre_ref, yim_ref, x_ref, o_ref):
    filt = jnp.dot(ic_ref[...], yre_ref[0], preferred_element_type=jnp.float32)
    filt = filt + jnp.dot(isn_ref[...], yim_ref[0], preferred_element_type=jnp.float32)
    o_ref[0] = x_ref[0] - filt


def kernel(batch_x, W1, b1, W2, b2, W3, b3):
    B = batch_x.shape[0]
    cth = jnp.asarray(_CTH)
    ctl = jnp.asarray(_CTL)
    sth = jnp.asarray(_STH)
    stl = jnp.asarray(_STL)
    ic = jnp.asarray(_IC)
    isn = jnp.asarray(_ISN)

    FH = FP // 4
    xre, xim = pl.pallas_call(
        _fwd_kernel,
        grid=(4, B),
        in_specs=[
            pl.BlockSpec((FH, T), lambda h, b: (h, 0)),
            pl.BlockSpec((FH, T), lambda h, b: (h, 0)),
            pl.BlockSpec((FH, T), lambda h, b: (h, 0)),
            pl.BlockSpec((FH, T), lambda h, b: (h, 0)),
            pl.BlockSpec((1, T, NCH), lambda h, b: (b, 0, 0)),
        ],
        out_specs=[
            pl.BlockSpec((1, FH, NCH), lambda h, b: (b, h, 0)),
            pl.BlockSpec((1, FH, NCH), lambda h, b: (b, h, 0)),
        ],
        out_shape=[jax.ShapeDtypeStruct((B, FP, NCH), jnp.float32)] * 2,
    )(cth, ctl, sth, stl, batch_x)

    yre, yim = pl.pallas_call(
        _topk_kernel,
        grid=(B,),
        in_specs=[
            pl.BlockSpec((1, FP, NCH), lambda b: (b, 0, 0)),
            pl.BlockSpec((1, FP, NCH), lambda b: (b, 0, 0)),
        ],
        out_specs=[
            pl.BlockSpec((1, FP, NCH), lambda b: (b, 0, 0)),
            pl.BlockSpec((1, FP, NCH), lambda b: (b, 0, 0)),
        ],
        out_shape=[jax.ShapeDtypeStruct((B, FP, NCH), jnp.bfloat16)] * 2,
        scratch_shapes=[pltpu.VMEM((FP, NCH), jnp.float32)],
    )(xre, xim)

    TH = T // 2
    out = pl.pallas_call(
        _inv_kernel,
        grid=(2, B),
        in_specs=[
            pl.BlockSpec((TH, FP), lambda h, b: (h, 0)),
            pl.BlockSpec((TH, FP), lambda h, b: (h, 0)),
            pl.BlockSpec((1, FP, NCH), lambda h, b: (b, 0, 0)),
            pl.BlockSpec((1, FP, NCH), lambda h, b: (b, 0, 0)),
            pl.BlockSpec((1, TH, NCH), lambda h, b: (b, h, 0)),
        ],
        out_specs=pl.BlockSpec((1, TH, NCH), lambda h, b: (b, h, 0)),
        out_shape=jax.ShapeDtypeStruct((B, T, NCH), jnp.float32),
    )(ic, isn, yre, yim, batch_x)
    return out
